# Initial kernel scaffold; baseline (speedup 1.0000x reference)
import jax
import jax.numpy as jnp
from jax.experimental import pallas as pl


def kernel(hidden_states, gate_weight, w1, w3, w2, shared_w1, shared_w3, shared_w2):
    raise NotImplementedError("write your pallas kernel here")


# speedup vs baseline: 1.1186x; 1.1186x over previous
import functools
import jax
import jax.numpy as jnp
from jax import lax
from jax.experimental import pallas as pl
from jax.experimental.pallas import tpu as pltpu

T = 2048
D = 1024
F = 512
E = 64
K = 2
C = 128
EC = E * C

NEG = -1e30


_RB = 256


def _router_body(x_ref, g_ref, meta_ref):
    xb = x_ref[...]
    gw = g_ref[...]
    logits = lax.dot_general(xb, gw, (((1,), (1,)), ((), ())),
                             preferred_element_type=jnp.float32)
    eio = lax.broadcasted_iota(jnp.int32, (_RB, E), 1)
    m1 = jnp.max(logits, axis=1, keepdims=True)
    i1 = jnp.min(jnp.where(logits == m1, eio, E), axis=1, keepdims=True)
    masked = jnp.where(eio == i1, NEG, logits)
    m2 = jnp.max(masked, axis=1, keepdims=True)
    i2 = jnp.min(jnp.where(masked == m2, eio, E), axis=1, keepdims=True)
    w1 = jax.nn.sigmoid(m1 - m2)
    w2 = 1.0 - w1
    cio = lax.broadcasted_iota(jnp.int32, (_RB, 128), 1)
    meta = jnp.where(cio == 0, i1.astype(jnp.float32),
           jnp.where(cio == 1, i2.astype(jnp.float32),
           jnp.where(cio == 2, w1,
           jnp.where(cio == 3, w2, 0.0))))
    meta_ref[...] = meta


def _router(x, gate_weight):
    return pl.pallas_call(
        _router_body,
        grid=(T // _RB,),
        in_specs=[
            pl.BlockSpec((_RB, D), lambda i: (i, 0)),
            pl.BlockSpec((E, D), lambda i: (0, 0)),
        ],
        out_specs=pl.BlockSpec((_RB, 128), lambda i: (i, 0)),
        out_shape=jax.ShapeDtypeStruct((T, 128), jnp.float32),
    )(x, gate_weight)


def _moe_body(buf_ref, w1_ref, w3_ref, w2_ref, out_ref):
    xb = buf_ref[...]
    g = lax.dot_general(xb, w1_ref[0], (((1,), (1,)), ((), ())),
                        preferred_element_type=jnp.float32)
    u = lax.dot_general(xb, w3_ref[0], (((1,), (1,)), ((), ())),
                        preferred_element_type=jnp.float32)
    h = g * jax.nn.sigmoid(g) * u
    out_ref[...] = lax.dot_general(h, w2_ref[0], (((1,), (1,)), ((), ())),
                                   preferred_element_type=jnp.float32)


def _moe_mlp(buf, w1, w3, w2):
    return pl.pallas_call(
        _moe_body,
        grid=(E,),
        in_specs=[
            pl.BlockSpec((C, D), lambda e: (e, 0)),
            pl.BlockSpec((1, F, D), lambda e: (e, 0, 0)),
            pl.BlockSpec((1, F, D), lambda e: (e, 0, 0)),
            pl.BlockSpec((1, D, F), lambda e: (e, 0, 0)),
        ],
        out_specs=pl.BlockSpec((C, D), lambda e: (e, 0)),
        out_shape=jax.ShapeDtypeStruct((EC, D), jnp.float32),
    )(buf, w1, w3, w2)


_SB = 256


def _shared_body(x_ref, r_ref, w1_ref, w3_ref, w2_ref, out_ref):
    xb = x_ref[...]
    g = lax.dot_general(xb, w1_ref[...], (((1,), (1,)), ((), ())),
                        preferred_element_type=jnp.float32)
    u = lax.dot_general(xb, w3_ref[...], (((1,), (1,)), ((), ())),
                        preferred_element_type=jnp.float32)
    h = g * jax.nn.sigmoid(g) * u
    out_ref[...] = r_ref[...] + lax.dot_general(
        h, w2_ref[...], (((1,), (1,)), ((), ())),
        preferred_element_type=jnp.float32)


def _shared_mlp(x, routed, sw1, sw3, sw2):
    return pl.pallas_call(
        _shared_body,
        grid=(T // _SB,),
        in_specs=[
            pl.BlockSpec((_SB, D), lambda i: (i, 0)),
            pl.BlockSpec((_SB, D), lambda i: (i, 0)),
            pl.BlockSpec((F, D), lambda i: (0, 0)),
            pl.BlockSpec((F, D), lambda i: (0, 0)),
            pl.BlockSpec((D, F), lambda i: (0, 0)),
        ],
        out_specs=pl.BlockSpec((_SB, D), lambda i: (i, 0)),
        out_shape=jax.ShapeDtypeStruct((T, D), jnp.float32),
    )(x, routed, sw1, sw3, sw2)


def kernel(hidden_states, gate_weight, w1, w3, w2, shared_w1, shared_w3,
           shared_w2):
    x = hidden_states
    meta = _router(x, gate_weight)

    pe = jnp.concatenate([meta[:, 0:1], meta[:, 1:2]], axis=1).astype(jnp.int32).reshape(-1)
    pw = jnp.concatenate([meta[:, 2:3], meta[:, 3:4]], axis=1).reshape(-1)
    oh = jax.nn.one_hot(pe, E, dtype=jnp.int32)
    excl = jnp.cumsum(oh, axis=0) - oh
    pos = jnp.take_along_axis(excl, pe[:, None], axis=1)[:, 0]
    valid = pos < C
    slot = jnp.clip(pe * C + pos, 0, EC - 1)
    pt = jnp.repeat(jnp.arange(T), K)
    disp = x[pt] * valid[:, None].astype(x.dtype)
    buf = jnp.zeros((EC, D), x.dtype).at[slot].add(disp)

    ebuf = _moe_mlp(buf, w1, w3, w2)

    wv = pw * valid.astype(pw.dtype)
    ypairs = ebuf[slot] * wv[:, None]
    routed = ypairs.reshape(T, K, D).sum(axis=1)

    return _shared_mlp(x, routed, shared_w1, shared_w3, shared_w2)
